# baseline (speedup 1.0000x reference)
import jax
import jax.numpy as jnp
from jax import lax
from jax.experimental import pallas as pl
from jax.experimental.pallas import tpu as pltpu
from jax.experimental.pallas import tpu_sc as plsc

N_FACTORS = 64
BATCH = 16384
NC = 2
NS = 16
NW = NC * NS
B_PER_W = BATCH // NW
IDX_CHUNK = 128
N_CHUNKS = B_PER_W // IDX_CHUNK
GROUPS = B_PER_W // 16


def _sc_body(data_hbm, uf_hbm, if_hbm, out_hbm,
             data_v, idx_u, idx_v, u_rows, v_rows, out_buf, sem):
    wid = lax.axis_index("s") * NC + lax.axis_index("c")

    pltpu.sync_copy(data_hbm.at[wid], data_v)

    lane2 = lax.iota(jnp.int32, 16) * 2
    for g in range(GROUPS):
        pos = lane2 + (32 * g)
        us = plsc.load_gather(data_v, [pos])
        it = plsc.load_gather(data_v, [pos + 1])
        idx_u[g // 8, pl.ds((g % 8) * 16, 16)] = us
        idx_v[g // 8, pl.ds((g % 8) * 16, 16)] = it

    copies = []
    for j in range(N_CHUNKS):
        dst = u_rows.at[pl.ds(j * IDX_CHUNK, IDX_CHUNK)]
        copies.append(pltpu.async_copy(uf_hbm.at[idx_u.at[j]], dst, sem))
    for j in range(N_CHUNKS):
        dst = v_rows.at[pl.ds(j * IDX_CHUNK, IDX_CHUNK)]
        copies.append(pltpu.async_copy(if_hbm.at[idx_v.at[j]], dst, sem))
    for c in copies:
        c.wait()

    lane = lax.iota(jnp.int32, 16)

    def group_body(g, _):
        rows = g * 16 + lane
        acc0 = jnp.zeros((16,), jnp.float32)
        acc1 = jnp.zeros((16,), jnp.float32)
        acc2 = jnp.zeros((16,), jnp.float32)
        acc3 = jnp.zeros((16,), jnp.float32)
        accs = [acc0, acc1, acc2, acc3]
        for d in range(N_FACTORS):
            col = jnp.full((16,), d, jnp.int32)
            u = plsc.load_gather(u_rows, [rows, col])
            v = plsc.load_gather(v_rows, [rows, col])
            accs[d % 4] = accs[d % 4] + u * v
        out_buf[g] = (accs[0] + accs[1]) + (accs[2] + accs[3])
        return 0

    lax.fori_loop(0, GROUPS, group_body, 0)

    pltpu.sync_copy(out_buf, out_hbm.at[pl.ds(wid * GROUPS, GROUPS)])


@jax.jit
def _mf_dot(data, user_factors, item_factors):
    mesh = plsc.VectorSubcoreMesh(
        core_axis_name="c", subcore_axis_name="s",
        num_cores=NC, num_subcores=NS)
    k = pl.kernel(
        _sc_body,
        out_type=jax.ShapeDtypeStruct((BATCH // 16, 16), jnp.float32),
        mesh=mesh,
        compiler_params=pltpu.CompilerParams(
            needs_layout_passes=False, use_tc_tiling_on_sc=False),
        scratch_types=[
            pltpu.VMEM((2 * B_PER_W,), jnp.int32),
            pltpu.VMEM((N_CHUNKS, IDX_CHUNK), jnp.int32),
            pltpu.VMEM((N_CHUNKS, IDX_CHUNK), jnp.int32),
            pltpu.VMEM((B_PER_W, N_FACTORS), jnp.float32),
            pltpu.VMEM((B_PER_W, N_FACTORS), jnp.float32),
            pltpu.VMEM((GROUPS, 16), jnp.float32),
            pltpu.SemaphoreType.DMA,
        ],
    )
    return k(data, user_factors, item_factors)


def kernel(data, user_factors, item_factors):
    pairs = data.astype(jnp.int32).reshape(NW, 2 * B_PER_W)
    out = _mf_dot(pairs, user_factors, item_factors)
    return out.reshape(BATCH)


# speedup vs baseline: 1.0191x; 1.0191x over previous
import jax
import jax.numpy as jnp
from jax import lax
from jax.experimental import pallas as pl
from jax.experimental.pallas import tpu as pltpu
from jax.experimental.pallas import tpu_sc as plsc

N_FACTORS = 64
BATCH = 16384
NC = 2
NS = 16
NW = NC * NS
B_PER_W = BATCH // NW
GROUPS = B_PER_W // 16


def _sc_body(users_hbm, items_hbm, uf_hbm, if_hbm, out_hbm,
             idx_u, idx_v, u_rows, v_rows, out_buf, sem):
    wid = lax.axis_index("s") * NC + lax.axis_index("c")

    pltpu.sync_copy(users_hbm.at[pl.ds(wid, 1)], idx_u)
    pltpu.sync_copy(items_hbm.at[pl.ds(wid, 1)], idx_v)

    cu = pltpu.async_copy(uf_hbm.at[idx_u.at[0]], u_rows, sem)
    cv = pltpu.async_copy(if_hbm.at[idx_v.at[0]], v_rows, sem)
    cu.wait()
    cv.wait()

    lane = lax.iota(jnp.int32, 16)

    def group_body(g, _):
        rows = g * 16 + lane
        acc0 = jnp.zeros((16,), jnp.float32)
        acc1 = jnp.zeros((16,), jnp.float32)
        acc2 = jnp.zeros((16,), jnp.float32)
        acc3 = jnp.zeros((16,), jnp.float32)
        accs = [acc0, acc1, acc2, acc3]
        for d in range(N_FACTORS):
            col = jnp.full((16,), d, jnp.int32)
            u = plsc.load_gather(u_rows, [rows, col])
            v = plsc.load_gather(v_rows, [rows, col])
            accs[d % 4] = accs[d % 4] + u * v
        out_buf[g] = (accs[0] + accs[1]) + (accs[2] + accs[3])
        return 0

    lax.fori_loop(0, GROUPS, group_body, 0)

    pltpu.sync_copy(out_buf, out_hbm.at[pl.ds(wid * GROUPS, GROUPS)])


@jax.jit
def _mf_dot(users, items, user_factors, item_factors):
    mesh = plsc.VectorSubcoreMesh(
        core_axis_name="c", subcore_axis_name="s",
        num_cores=NC, num_subcores=NS)
    k = pl.kernel(
        _sc_body,
        out_type=jax.ShapeDtypeStruct((BATCH // 16, 16), jnp.float32),
        mesh=mesh,
        compiler_params=pltpu.CompilerParams(
            needs_layout_passes=False, use_tc_tiling_on_sc=False,
            disable_bounds_checks=True),
        scratch_types=[
            pltpu.VMEM((1, B_PER_W), jnp.int32),
            pltpu.VMEM((1, B_PER_W), jnp.int32),
            pltpu.VMEM((B_PER_W, N_FACTORS), jnp.float32),
            pltpu.VMEM((B_PER_W, N_FACTORS), jnp.float32),
            pltpu.VMEM((GROUPS, 16), jnp.float32),
            pltpu.SemaphoreType.DMA,
        ],
    )
    return k(users, items, user_factors, item_factors)


def kernel(data, user_factors, item_factors):
    users = data[:, 0].astype(jnp.int32).reshape(NW, B_PER_W)
    items = data[:, 1].astype(jnp.int32).reshape(NW, B_PER_W)
    out = _mf_dot(users, items, user_factors, item_factors)
    return out.reshape(BATCH)


# speedup vs baseline: 1.0327x; 1.0133x over previous
import jax
import jax.numpy as jnp
from jax import lax
from jax.experimental import pallas as pl
from jax.experimental.pallas import tpu as pltpu
from jax.experimental.pallas import tpu_sc as plsc

N_FACTORS = 64
BATCH = 16384
NC = 2
NS = 16
NW = NC * NS
B_PER_W = BATCH // NW
CHUNK = 128
N_CHUNKS = B_PER_W // CHUNK
G_PER_CHUNK = CHUNK // 16
GROUPS = B_PER_W // 16


def _sc_body(users_hbm, items_hbm, uf_hbm, if_hbm, out_hbm,
             idx_u, idx_v, pair_u, pair_v,
             u_buf0, u_buf1, v_buf0, v_buf1, out_buf,
             sem_u0, sem_u1, sem_v0, sem_v1):
    wid = lax.axis_index("s") * NC + lax.axis_index("c")
    base = wid * B_PER_W

    pltpu.sync_copy(users_hbm.at[pl.ds(base, B_PER_W)], idx_u)
    pltpu.sync_copy(items_hbm.at[pl.ds(base, B_PER_W)], idx_v)

    def pair_body(g, _):
        s = g * 16
        pair_u[pl.ds(s, 16)] = lax.shift_right_logical(idx_u[pl.ds(s, 16)], 1)
        pair_v[pl.ds(s, 16)] = lax.shift_right_logical(idx_v[pl.ds(s, 16)], 1)
        return 0
    lax.fori_loop(0, GROUPS, pair_body, 0)

    u_bufs = [u_buf0, u_buf1]
    v_bufs = [v_buf0, v_buf1]
    sems_u = [sem_u0, sem_u1]
    sems_v = [sem_v0, sem_v1]

    def fire(c):
        b = c % 2
        cu = pltpu.async_copy(
            uf_hbm.at[pair_u.at[pl.ds(c * CHUNK, CHUNK)]], u_bufs[b],
            sems_u[b])
        cv = pltpu.async_copy(
            if_hbm.at[pair_v.at[pl.ds(c * CHUNK, CHUNK)]], v_bufs[b],
            sems_v[b])
        return cu, cv

    lane = lax.iota(jnp.int32, 16)
    pend = fire(0)

    for c in range(N_CHUNKS):
        b = c % 2
        cu, cv = pend
        cu.wait()
        cv.wait()
        if c + 1 < N_CHUNKS:
            pend = fire(c + 1)
        u_rows = u_bufs[b]
        v_rows = v_bufs[b]

        def gbody(g, _, c=c, u_rows=u_rows, v_rows=v_rows):
            rows = g * 16 + lane
            gslice = pl.ds(c * CHUNK + g * 16, 16)
            hu = lax.shift_left(jnp.bitwise_and(idx_u[gslice], 1), 6)
            hv = lax.shift_left(jnp.bitwise_and(idx_v[gslice], 1), 6)
            acc0 = jnp.zeros((16,), jnp.float32)
            acc1 = jnp.zeros((16,), jnp.float32)
            acc2 = jnp.zeros((16,), jnp.float32)
            acc3 = jnp.zeros((16,), jnp.float32)
            accs = [acc0, acc1, acc2, acc3]
            for d in range(N_FACTORS):
                u = plsc.load_gather(u_rows, [rows, hu + d])
                v = plsc.load_gather(v_rows, [rows, hv + d])
                accs[d % 4] = accs[d % 4] + u * v
            out_buf[c * G_PER_CHUNK + g] = (accs[0] + accs[1]) + (accs[2] + accs[3])
            return 0

        lax.fori_loop(0, G_PER_CHUNK, gbody, 0)

    pltpu.sync_copy(out_buf, out_hbm.at[pl.ds(wid * GROUPS, GROUPS)])


@jax.jit
def _mf_dot(users, items, user_factors, item_factors):
    mesh = plsc.VectorSubcoreMesh(
        core_axis_name="c", subcore_axis_name="s",
        num_cores=NC, num_subcores=NS)
    k = pl.kernel(
        _sc_body,
        out_type=jax.ShapeDtypeStruct((BATCH // 16, 16), jnp.float32),
        mesh=mesh,
        compiler_params=pltpu.CompilerParams(
            needs_layout_passes=False, use_tc_tiling_on_sc=False,
            disable_bounds_checks=True),
        scratch_types=[
            pltpu.VMEM((B_PER_W,), jnp.int32),
            pltpu.VMEM((B_PER_W,), jnp.int32),
            pltpu.VMEM((B_PER_W,), jnp.int32),
            pltpu.VMEM((B_PER_W,), jnp.int32),
            pltpu.VMEM((CHUNK, 2 * N_FACTORS), jnp.float32),
            pltpu.VMEM((CHUNK, 2 * N_FACTORS), jnp.float32),
            pltpu.VMEM((CHUNK, 2 * N_FACTORS), jnp.float32),
            pltpu.VMEM((CHUNK, 2 * N_FACTORS), jnp.float32),
            pltpu.VMEM((GROUPS, 16), jnp.float32),
            pltpu.SemaphoreType.DMA,
            pltpu.SemaphoreType.DMA,
            pltpu.SemaphoreType.DMA,
            pltpu.SemaphoreType.DMA,
        ],
    )
    return k(users, items, user_factors, item_factors)


def kernel(data, user_factors, item_factors):
    users = data[:, 0].astype(jnp.int32)
    items = data[:, 1].astype(jnp.int32)
    uf2 = user_factors.reshape(50000, 2 * N_FACTORS)
    if2 = item_factors.reshape(50000, 2 * N_FACTORS)
    out = _mf_dot(users, items, uf2, if2)
    return out.reshape(BATCH)
